# baseline (speedup 1.0000x reference)
import functools

import jax
import jax.numpy as jnp
from jax import lax
from jax.experimental import pallas as pl
from jax.experimental.pallas import tpu as pltpu
from jax.experimental.pallas import tpu_sc as plsc


def _tidx_body(H, W, HB, off0_ref, off1_ref, T_ref):
    i = pl.program_id(1)
    hb = lax.broadcasted_iota(jnp.int32, (HB, W), 0) + i * HB
    wb = lax.broadcasted_iota(jnp.int32, (HB, W), 1)
    o0 = (off0_ref[0] * 0.5).astype(jnp.int32)
    o1 = (off1_ref[0] * 0.5).astype(jnp.int32)
    t0 = jnp.clip(hb + o0, 0, H - 1)
    t1 = jnp.clip(wb + o1, 0, W - 1)
    T_ref[0] = t0 * W + t1


def _make_tidx(N, C, H, W, HB, interpret=False):
    return pl.pallas_call(
        functools.partial(_tidx_body, H, W, HB),
        grid=(N, H // HB),
        in_specs=[
            pl.BlockSpec((1, HB, W), lambda n, i: (n, i, 0)),
            pl.BlockSpec((1, HB, W), lambda n, i: (n, i, 0)),
        ],
        out_specs=pl.BlockSpec((1, HB, W), lambda n, i: (n, i, 0)),
        out_shape=jax.ShapeDtypeStruct((N, H, W), jnp.int32),
        interpret=interpret,
    )


def _xpose_body(f_ref, fpx_ref):
    ft = jnp.swapaxes(f_ref[0], 0, 1)
    K_, C_ = ft.shape
    fpx_ref[...] = jnp.concatenate(
        [ft, jnp.zeros((K_, 128 - C_), jnp.float32)], axis=1)


def _make_xpose(N, C, H, W, HB, interpret=False):
    HW = H * W
    K = HB * W
    return pl.pallas_call(
        _xpose_body,
        grid=(N, H // HB),
        in_specs=[
            pl.BlockSpec((1, C, K), lambda n, i: (n, 0, i)),
        ],
        out_specs=pl.BlockSpec((K, 128), lambda n, i: (n * (HW // K) + i, 0)),
        out_shape=jax.ShapeDtypeStruct((N * HW, 128), jnp.float32),
        interpret=interpret,
    )


def _blend_body(C, f_ref, g_ref, w_ref, o_ref):
    w = w_ref[0]
    g = jnp.swapaxes(g_ref[..., :C], 0, 1)
    o_ref[0] = f_ref[0] * (1.0 - w) + g * w


def _blend_body_alias(C, prev_ref, f_ref, g_ref, w_ref, o_ref):
    del prev_ref
    _blend_body(C, f_ref, g_ref, w_ref, o_ref)


def _make_blend_n(N, C, H, W, HB, nfix, aliased, interpret=False):
    HW = H * W
    K = HB * W
    in_specs = [
        pl.BlockSpec((1, C, K), lambda i: (nfix, 0, i)),
        pl.BlockSpec((K, 128), lambda i: (i, 0)),
        pl.BlockSpec((1, 1, K), lambda i: (nfix, 0, i)),
    ]
    body = functools.partial(_blend_body, C)
    alias = {}
    if aliased:
        in_specs = [pl.BlockSpec((1, 8, K), lambda i: (nfix, 0, i))] + in_specs
        body = functools.partial(_blend_body_alias, C)
        alias = {0: 0}
    return pl.pallas_call(
        body,
        grid=(H // HB,),
        in_specs=in_specs,
        out_specs=pl.BlockSpec((1, C, K), lambda i: (nfix, 0, i)),
        out_shape=jax.ShapeDtypeStruct((N, C, HW), jnp.float32),
        input_output_aliases=alias,
        interpret=interpret,
    )


_NTILE = 16
_MESH = dict(core_axis_name="c", subcore_axis_name="s",
             num_cores=2, num_subcores=_NTILE)


def _make_sc_scatter(N, C, H, W, interpret=False):
    HW = H * W
    NTILE = _NTILE
    RNG = HW // NTILE
    RPT = H // NTILE
    B = 4
    WCHR = 4
    WCH = WCHR * W
    OLCAP = RNG
    OBLK = 256
    assert RNG % 16 == 0 and W % 16 == 0
    assert RPT % WCHR == 0 and B % WCHR == 0

    mesh = plsc.VectorSubcoreMesh(**_MESH)

    @functools.partial(
        pl.kernel,
        out_type=jax.ShapeDtypeStruct((N * HW,), jnp.int32),
        mesh=mesh,
        scratch_types=[
            pltpu.VMEM((RNG,), jnp.int32),
            pltpu.VMEM((WCH,), jnp.int32),
            pltpu.VMEM((OLCAP,), jnp.int32),
            pltpu.VMEM((OLCAP,), jnp.int32),
            pltpu.SMEM((NTILE,), jnp.int32),
            pltpu.VMEM_SHARED((NTILE, 2, OLCAP), jnp.int32),
        ],
        compiler_params=pltpu.CompilerParams(
            needs_layout_passes=False, use_tc_tiling_on_sc=True),
        interpret=interpret,
    )
    def sc_scatter(T_hbm, g_hbm, m_ref, tbuf, olT, olj, cnt_smem, ol_shared):
        c = lax.axis_index("c")
        s = lax.axis_index("s")
        base_t = s * RNG
        lane = lax.iota(jnp.int32, 16)

        def init_body(i, _):
            m_ref[pl.ds(i * 16, 16)] = jnp.full((16,), -1, jnp.int32)
            return _
        lax.fori_loop(0, RNG // 16, init_body, None)

        def cmpx(Tv, jv, extra_mask):
            plv = Tv - base_t
            inr = extra_mask & (plv >= 0) & (plv < RNG)
            plc = jnp.clip(plv, 0, RNG - 1)
            cur0 = plsc.load_gather(m_ref, [plc])
            need0 = inr & (jv > cur0)

            def cond(need):
                return jnp.any(need)

            def body(need):
                plsc.store_scatter(m_ref, [plc], jv, mask=need)
                cur = plsc.load_gather(m_ref, [plc])
                return inr & (jv > cur)

            lax.while_loop(cond, body, need0)

        r0 = jnp.maximum(s * RPT - B, 0)
        r1 = jnp.minimum((s + 1) * RPT + B, H)
        nwch = (r1 - r0) * W // WCH
        wstart = c * HW + r0 * W

        def zcnt(i, _):
            cnt_smem[i] = 0
            return _
        lax.fori_loop(0, NTILE, zcnt, None)
        plsc.subcore_barrier()

        def chunk_body(q, cnt):
            wbase = r0 * W + q * WCH
            pltpu.sync_copy(T_hbm.at[pl.ds(c * HW + wbase, WCH)], tbuf)

            def vec_body(i, cnt):
                Tv = tbuf[pl.ds(i * 16, 16)]
                j0 = wbase + i * 16
                jv = j0 + lane
                cmpx(Tv, jv, jnp.full((16,), True))
                strict = (j0 >= base_t) & (j0 < base_t + RNG)
                jh = j0 // W
                lo = (jh - B) * W
                hi = (jh + B + 1) * W
                outl = strict & ((Tv < lo) | (Tv >= hi))

                def append(cnt):
                    plsc.store_compressed(olT.at[pl.ds(cnt, 16)], Tv,
                                          mask=outl)
                    plsc.store_compressed(olj.at[pl.ds(cnt, 16)], jv,
                                          mask=outl)
                    return cnt + jnp.sum(outl.astype(jnp.int32))

                return lax.cond(jnp.any(outl), append, lambda x: x, cnt)
            return lax.fori_loop(0, WCH // 16, vec_body, cnt)
        cnt = lax.fori_loop(0, nwch, chunk_body, jnp.int32(0))

        pltpu.sync_copy(olT, ol_shared.at[s, 0])
        pltpu.sync_copy(olj, ol_shared.at[s, 1])
        for dst in range(NTILE):
            plsc.fetch_and_add(cnt_smem.at[s], cnt, subcore_id=dst)
        plsc.subcore_barrier()

        for src in range(NTILE):
            cnt_src = cnt_smem[src]

            def blk_cond(bk):
                return bk * OBLK < cnt_src

            def blk_body(bk):
                pltpu.sync_copy(ol_shared.at[src, 0, pl.ds(bk * OBLK, OBLK)],
                                olT.at[pl.ds(0, OBLK)])
                pltpu.sync_copy(ol_shared.at[src, 1, pl.ds(bk * OBLK, OBLK)],
                                olj.at[pl.ds(0, OBLK)])

                def v_body(i, _):
                    valid = (bk * OBLK + i * 16 + lane) < cnt_src
                    Tv = olT[pl.ds(i * 16, 16)]
                    jv = olj[pl.ds(i * 16, 16)]
                    cmpx(Tv, jv, valid)
                    return _
                lax.fori_loop(0, OBLK // 16, v_body, None)
                return bk + 1
            lax.while_loop(blk_cond, blk_body, jnp.int32(0))

        def g_body(i, _):
            v = m_ref[pl.ds(i * 16, 16)]
            pv = base_t + i * 16 + lane
            je = jnp.where(v >= 0, v, pv)
            g = (je % W) * W + je // W + c * HW
            m_ref[pl.ds(i * 16, 16)] = g
            return _
        lax.fori_loop(0, RNG // 16, g_body, None)
        pltpu.sync_copy(m_ref, g_hbm.at[pl.ds(c * HW + base_t, RNG)])

    return sc_scatter


def _make_sc_gather(N, C, H, W, nfix, interpret=False):
    HW = H * W
    NW = 2 * _NTILE
    RNG = HW // NW
    CB = min(128, RNG)
    NBUF = 4
    LAG = NBUF - 1
    assert RNG % CB == 0

    mesh = plsc.VectorSubcoreMesh(**_MESH)

    @functools.partial(
        pl.kernel,
        out_type=jax.ShapeDtypeStruct((HW, 128), jnp.float32),
        mesh=mesh,
        scratch_types=[
            pltpu.VMEM((RNG,), jnp.int32),
            pltpu.VMEM((NBUF, CB, 128), jnp.float32),
            pltpu.SemaphoreType.DMA,
            pltpu.SemaphoreType.DMA,
            pltpu.SemaphoreType.DMA,
            pltpu.SemaphoreType.DMA,
            pltpu.SemaphoreType.DMA,
            pltpu.SemaphoreType.DMA,
            pltpu.SemaphoreType.DMA,
            pltpu.SemaphoreType.DMA,
        ],
        compiler_params=pltpu.CompilerParams(
            needs_layout_passes=False, use_tc_tiling_on_sc=True),
        interpret=interpret,
    )
    def sc_gather(g_hbm, fpx_hbm, gat_hbm, g_ref, rbufs,
                  gs0, gs1, gs2, gs3, ss0, ss1, ss2, ss3):
        c = lax.axis_index("c")
        s = lax.axis_index("s")
        w = c * _NTILE + s
        gsems = (gs0, gs1, gs2, gs3)
        ssems = (ss0, ss1, ss2, ss3)
        outbase = w * RNG
        nck = RNG // CB
        pltpu.sync_copy(g_hbm.at[pl.ds(nfix * HW + outbase, RNG)], g_ref)

        def start_gather(k):
            return pltpu.async_copy(
                fpx_hbm.at[g_ref.at[pl.ds(k * CB, CB)]],
                rbufs.at[k % NBUF], gsems[k % NBUF])

        def start_scatter(k):
            return pltpu.async_copy(
                rbufs.at[k % NBUF],
                gat_hbm.at[pl.ds(outbase + k * CB, CB)], ssems[k % NBUF])

        gdesc, sdesc = {}, {}
        for k in range(min(LAG, nck)):
            gdesc[k] = start_gather(k)
        for k in range(nck):
            if k + LAG < nck:
                if k + LAG >= NBUF:
                    sdesc[k + LAG - NBUF].wait()
                gdesc[k + LAG] = start_gather(k + LAG)
            gdesc[k].wait()
            sdesc[k] = start_scatter(k)
        for k in range(max(0, nck - NBUF), nck):
            sdesc[k].wait()

    return sc_gather


def _run(feature, offset, weight, interpret=False):
    N, C, H, W = feature.shape
    HW = H * W
    HB = 8
    f3 = feature.reshape(N, C, HW)
    off0 = offset[..., 0]
    off1 = offset[..., 1]
    T = _make_tidx(N, C, H, W, HB, interpret)(off0, off1)
    g = _make_sc_scatter(N, C, H, W, interpret)(T.reshape(N * HW))
    fpx = _make_xpose(N, C, H, W, HB, interpret)(f3)
    w3 = weight.reshape(N, 1, HW)
    out = None
    for n in range(N):
        gat_n = _make_sc_gather(N, C, H, W, n, interpret)(g, fpx)
        if out is None:
            out = _make_blend_n(N, C, H, W, HB, n, False, interpret)(
                f3, gat_n, w3)
        else:
            out = _make_blend_n(N, C, H, W, HB, n, True, interpret)(
                out, f3, gat_n, w3)
    return out.reshape(N, C, H, W)


def kernel(feature, offset, weight):
    return _run(feature, offset, weight)


# speedup vs baseline: 1.0021x; 1.0021x over previous
import functools

import jax
import jax.numpy as jnp
from jax import lax
from jax.experimental import pallas as pl
from jax.experimental.pallas import tpu as pltpu
from jax.experimental.pallas import tpu_sc as plsc


def _tidx_body(H, W, HB, off0_ref, off1_ref, T_ref):
    i = pl.program_id(1)
    hb = lax.broadcasted_iota(jnp.int32, (HB, W), 0) + i * HB
    wb = lax.broadcasted_iota(jnp.int32, (HB, W), 1)
    o0 = (off0_ref[0] * 0.5).astype(jnp.int32)
    o1 = (off1_ref[0] * 0.5).astype(jnp.int32)
    t0 = jnp.clip(hb + o0, 0, H - 1)
    t1 = jnp.clip(wb + o1, 0, W - 1)
    T_ref[0] = t0 * W + t1


def _make_tidx(N, C, H, W, HB, interpret=False):
    return pl.pallas_call(
        functools.partial(_tidx_body, H, W, HB),
        grid=(N, H // HB),
        in_specs=[
            pl.BlockSpec((1, HB, W), lambda n, i: (n, i, 0)),
            pl.BlockSpec((1, HB, W), lambda n, i: (n, i, 0)),
        ],
        out_specs=pl.BlockSpec((1, HB, W), lambda n, i: (n, i, 0)),
        out_shape=jax.ShapeDtypeStruct((N, H, W), jnp.int32),
        interpret=interpret,
    )


def _xpose_body(f_ref, fpx_ref):
    ft = jnp.swapaxes(f_ref[0], 0, 1)
    K_, C_ = ft.shape
    fpx_ref[...] = jnp.concatenate(
        [ft, jnp.zeros((K_, 128 - C_), jnp.float32)], axis=1)


def _make_xpose(N, C, H, W, HB, interpret=False):
    HW = H * W
    K = HB * W
    return pl.pallas_call(
        _xpose_body,
        grid=(N, H // HB),
        in_specs=[
            pl.BlockSpec((1, C, K), lambda n, i: (n, 0, i)),
        ],
        out_specs=pl.BlockSpec((K, 128), lambda n, i: (n * (HW // K) + i, 0)),
        out_shape=jax.ShapeDtypeStruct((N * HW, 128), jnp.float32),
        interpret=interpret,
    )


def _blend_body(C, f_ref, g_ref, w_ref, o_ref):
    w = w_ref[0]
    g = jnp.swapaxes(g_ref[..., :C], 0, 1)
    o_ref[0] = f_ref[0] * (1.0 - w) + g * w


def _blend_body_alias(C, prev_ref, f_ref, g_ref, w_ref, o_ref):
    del prev_ref
    _blend_body(C, f_ref, g_ref, w_ref, o_ref)


def _make_blend_n(N, C, H, W, HB, nfix, aliased, interpret=False):
    HW = H * W
    K = HB * W
    in_specs = [
        pl.BlockSpec((1, C, K), lambda i: (nfix, 0, i)),
        pl.BlockSpec((K, 128), lambda i: (i, 0)),
        pl.BlockSpec((1, 1, K), lambda i: (nfix, 0, i)),
    ]
    body = functools.partial(_blend_body, C)
    alias = {}
    if aliased:
        in_specs = [pl.BlockSpec((1, 8, K), lambda i: (nfix, 0, i))] + in_specs
        body = functools.partial(_blend_body_alias, C)
        alias = {0: 0}
    return pl.pallas_call(
        body,
        grid=(H // HB,),
        in_specs=in_specs,
        out_specs=pl.BlockSpec((1, C, K), lambda i: (nfix, 0, i)),
        out_shape=jax.ShapeDtypeStruct((N, C, HW), jnp.float32),
        input_output_aliases=alias,
        interpret=interpret,
    )


_NTILE = 16
_MESH = dict(core_axis_name="c", subcore_axis_name="s",
             num_cores=2, num_subcores=_NTILE)


def _make_sc_scatter(N, C, H, W, interpret=False):
    HW = H * W
    NTILE = _NTILE
    RNG = HW // NTILE
    RPT = H // NTILE
    B = 4
    WCHR = 4
    WCH = WCHR * W
    OLCAP = RNG
    OBLK = 256
    assert RNG % 16 == 0 and W % 16 == 0
    assert RPT % WCHR == 0 and B % WCHR == 0

    mesh = plsc.VectorSubcoreMesh(**_MESH)

    @functools.partial(
        pl.kernel,
        out_type=jax.ShapeDtypeStruct((N * HW,), jnp.int32),
        mesh=mesh,
        scratch_types=[
            pltpu.VMEM((RNG,), jnp.int32),
            pltpu.VMEM((WCH,), jnp.int32),
            pltpu.VMEM((OLCAP,), jnp.int32),
            pltpu.VMEM((OLCAP,), jnp.int32),
            pltpu.SMEM((NTILE,), jnp.int32),
            pltpu.VMEM_SHARED((NTILE, 2, OLCAP), jnp.int32),
        ],
        compiler_params=pltpu.CompilerParams(
            needs_layout_passes=False, use_tc_tiling_on_sc=True),
        interpret=interpret,
    )
    def sc_scatter(T_hbm, g_hbm, m_ref, tbuf, olT, olj, cnt_smem, ol_shared):
        c = lax.axis_index("c")
        s = lax.axis_index("s")
        base_t = s * RNG
        lane = lax.iota(jnp.int32, 16)

        def init_body(i, _):
            m_ref[pl.ds(i * 16, 16)] = jnp.full((16,), -1, jnp.int32)
            return _
        lax.fori_loop(0, RNG // 16, init_body, None)

        def cmpx(Tv, jv, extra_mask):
            plv = Tv - base_t
            inr = extra_mask & (plv >= 0) & (plv < RNG)
            plc = jnp.clip(plv, 0, RNG - 1)
            cur0 = plsc.load_gather(m_ref, [plc])
            need0 = inr & (jv > cur0)

            def cond(need):
                return jnp.any(need)

            def body(need):
                plsc.store_scatter(m_ref, [plc], jv, mask=need)
                cur = plsc.load_gather(m_ref, [plc])
                return inr & (jv > cur)

            lax.while_loop(cond, body, need0)

        r0 = jnp.maximum(s * RPT - B, 0)
        r1 = jnp.minimum((s + 1) * RPT + B, H)
        nwch = (r1 - r0) * W // WCH
        def zcnt(i, _):
            cnt_smem[i] = 0
            return _
        lax.fori_loop(0, NTILE, zcnt, None)
        plsc.subcore_barrier()

        def chunk_body(q, cnt):
            wbase = r0 * W + q * WCH
            pltpu.sync_copy(T_hbm.at[pl.ds(c * HW + wbase, WCH)], tbuf)

            def vec_body(i, cnt):
                Tv = tbuf[pl.ds(i * 16, 16)]
                j0 = wbase + i * 16
                jv = j0 + lane
                cmpx(Tv, jv, jnp.full((16,), True))
                strict = (j0 >= base_t) & (j0 < base_t + RNG)
                jh = j0 // W
                lo = (jh - B) * W
                hi = (jh + B + 1) * W
                outl = strict & ((Tv < lo) | (Tv >= hi))

                def append(cnt):
                    plsc.store_compressed(olT.at[pl.ds(cnt, 16)], Tv,
                                          mask=outl)
                    plsc.store_compressed(olj.at[pl.ds(cnt, 16)], jv,
                                          mask=outl)
                    return cnt + jnp.sum(outl.astype(jnp.int32))

                return lax.cond(jnp.any(outl), append, lambda x: x, cnt)
            return lax.fori_loop(0, WCH // 16, vec_body, cnt)
        cnt = lax.fori_loop(0, nwch, chunk_body, jnp.int32(0))

        pltpu.sync_copy(olT, ol_shared.at[s, 0])
        pltpu.sync_copy(olj, ol_shared.at[s, 1])
        for dst in range(NTILE):
            plsc.fetch_and_add(cnt_smem.at[s], cnt, subcore_id=dst)
        plsc.subcore_barrier()

        for src in range(NTILE):
            cnt_src = cnt_smem[src]

            def blk_cond(bk):
                return bk * OBLK < cnt_src

            def blk_body(bk):
                pltpu.sync_copy(ol_shared.at[src, 0, pl.ds(bk * OBLK, OBLK)],
                                olT.at[pl.ds(0, OBLK)])
                pltpu.sync_copy(ol_shared.at[src, 1, pl.ds(bk * OBLK, OBLK)],
                                olj.at[pl.ds(0, OBLK)])

                def v_body(i, _):
                    valid = (bk * OBLK + i * 16 + lane) < cnt_src
                    Tv = olT[pl.ds(i * 16, 16)]
                    jv = olj[pl.ds(i * 16, 16)]
                    cmpx(Tv, jv, valid)
                    return _
                lax.fori_loop(0, OBLK // 16, v_body, None)
                return bk + 1
            lax.while_loop(blk_cond, blk_body, jnp.int32(0))

        def g_body(i, _):
            v = m_ref[pl.ds(i * 16, 16)]
            pv = base_t + i * 16 + lane
            je = jnp.where(v >= 0, v, pv)
            g = (je % W) * W + je // W + c * HW
            m_ref[pl.ds(i * 16, 16)] = g
            return _
        lax.fori_loop(0, RNG // 16, g_body, None)
        pltpu.sync_copy(m_ref, g_hbm.at[pl.ds(c * HW + base_t, RNG)])

    return sc_scatter


def _make_sc_gather(N, C, H, W, nfix, interpret=False):
    HW = H * W
    NW = 2 * _NTILE
    RNG = HW // NW
    CB = min(128, RNG)
    NBUF = 4
    LAG = NBUF - 1
    assert RNG % CB == 0

    mesh = plsc.VectorSubcoreMesh(**_MESH)

    @functools.partial(
        pl.kernel,
        out_type=jax.ShapeDtypeStruct((HW, 128), jnp.float32),
        mesh=mesh,
        scratch_types=[
            pltpu.VMEM((RNG,), jnp.int32),
            pltpu.VMEM((NBUF, CB, 128), jnp.float32),
            pltpu.SemaphoreType.DMA,
            pltpu.SemaphoreType.DMA,
            pltpu.SemaphoreType.DMA,
            pltpu.SemaphoreType.DMA,
            pltpu.SemaphoreType.DMA,
            pltpu.SemaphoreType.DMA,
            pltpu.SemaphoreType.DMA,
            pltpu.SemaphoreType.DMA,
        ],
        compiler_params=pltpu.CompilerParams(
            needs_layout_passes=False, use_tc_tiling_on_sc=True),
        interpret=interpret,
    )
    def sc_gather(g_hbm, fpx_hbm, gat_hbm, g_ref, rbufs,
                  gs0, gs1, gs2, gs3, ss0, ss1, ss2, ss3):
        c = lax.axis_index("c")
        s = lax.axis_index("s")
        w = c * _NTILE + s
        gsems = (gs0, gs1, gs2, gs3)
        ssems = (ss0, ss1, ss2, ss3)
        outbase = w * RNG
        nck = RNG // CB
        pltpu.sync_copy(g_hbm.at[pl.ds(nfix * HW + outbase, RNG)], g_ref)

        def start_gather(k):
            return pltpu.async_copy(
                fpx_hbm.at[g_ref.at[pl.ds(k * CB, CB)]],
                rbufs.at[k % NBUF], gsems[k % NBUF])

        def start_scatter(k):
            return pltpu.async_copy(
                rbufs.at[k % NBUF],
                gat_hbm.at[pl.ds(outbase + k * CB, CB)], ssems[k % NBUF])

        gdesc, sdesc = {}, {}
        for k in range(min(LAG, nck)):
            gdesc[k] = start_gather(k)
        for k in range(nck):
            if k + LAG < nck:
                if k + LAG >= NBUF:
                    sdesc[k + LAG - NBUF].wait()
                gdesc[k + LAG] = start_gather(k + LAG)
            gdesc[k].wait()
            sdesc[k] = start_scatter(k)
        for k in range(max(0, nck - NBUF), nck):
            sdesc[k].wait()

    return sc_gather


def _run(feature, offset, weight, interpret=False):
    N, C, H, W = feature.shape
    HW = H * W
    HB = 8
    f3 = feature.reshape(N, C, HW)
    off0 = offset[..., 0]
    off1 = offset[..., 1]
    T = _make_tidx(N, C, H, W, HB, interpret)(off0, off1)
    g = _make_sc_scatter(N, C, H, W, interpret)(T.reshape(N * HW))
    fpx = _make_xpose(N, C, H, W, HB, interpret)(f3)
    w3 = weight.reshape(N, 1, HW)
    out = None
    for n in range(N):
        gat_n = _make_sc_gather(N, C, H, W, n, interpret)(g, fpx)
        if out is None:
            out = _make_blend_n(N, C, H, W, HB, n, False, interpret)(
                f3, gat_n, w3)
        else:
            out = _make_blend_n(N, C, H, W, HB, n, True, interpret)(
                out, f3, gat_n, w3)
    return out.reshape(N, C, H, W)


def kernel(feature, offset, weight):
    return _run(feature, offset, weight)
